# baseline (speedup 1.0000x reference)
import functools

import jax
import jax.numpy as jnp
from jax import lax
from jax.experimental import pallas as pl
from jax.experimental.pallas import tpu as pltpu
from jax.experimental.pallas import tpu_sc as plsc

VOCAB = 1000000
D = 64
BATCH = 4096
SEQ = 200
B = BATCH * SEQ
PAD_ID = 0

NC = 2
NS = 16
L = 16
NW = NC * NS
BPW = B // NW
IPG = 128
CS = 1024
C = 512
G = C // IPG
GS = CS // IPG
CHUNKS = BPW // CS

_mesh = plsc.VectorSubcoreMesh(core_axis_name="c", subcore_axis_name="s")


@functools.partial(
    pl.kernel,
    out_type=jax.ShapeDtypeStruct((SEQ, D, BATCH), jnp.float32),
    mesh=_mesh,
    scratch_types=[
        pltpu.VMEM((GS, IPG), jnp.int32),
        pltpu.VMEM((GS, IPG), jnp.int32),
        pltpu.VMEM((C, IPG), jnp.float32),
        pltpu.VMEM((D, C), jnp.float32),
        pltpu.SemaphoreType.DMA,
    ],
    compiler_params=pltpu.CompilerParams(needs_layout_passes=False),
)
def _embed_lookup(ids_hbm, table_hbm, out_hbm, idx_v, idx2_v, rows2_v, trows_v, sem):
    wid = lax.axis_index("s") * NC + lax.axis_index("c")
    base = wid * BPW

    def chunk_body(k, carry):
        flat0 = base + k * CS
        s = flat0 >> 12
        b0 = pl.multiple_of(flat0 & (BATCH - 1), CS)
        bblk = pl.multiple_of(b0 >> 7, GS)

        pltpu.sync_copy(ids_hbm.at[s, pl.ds(bblk, GS)], idx_v)

        for j in range(GS):
            for t in range(IPG // L):
                sl = pl.ds(t * L, L)
                idv = idx_v[j, sl]
                hv = jnp.where(idv >= VOCAB // 2, 1, 0)
                idx2_v[j, sl] = idv - hv * (VOCAB // 2)

        for half in range(CS // C):
            copies = [
                pltpu.async_copy(
                    table_hbm.at[idx2_v.at[half * G + j]],
                    rows2_v.at[pl.ds(j * IPG, IPG)],
                    sem,
                )
                for j in range(G)
            ]
            for cp in copies:
                cp.wait()

            def rep_body(j, carry2):
                hv, mv, riota = [], [], []
                for t in range(IPG // L):
                    idv = idx_v[half * G + j, pl.ds(t * L, L)]
                    hv.append(jnp.where(idv >= VOCAB // 2, 1, 0))
                    mv.append(
                        jnp.where(idv == PAD_ID, 0.0, 1.0).astype(jnp.float32)
                    )
                    riota.append(j * IPG + t * L + lax.iota(jnp.int32, L))
                for e in range(D):
                    for t in range(IPG // L):
                        g = plsc.load_gather(rows2_v, [riota[t], hv[t] + 2 * e])
                        trows_v[e, pl.ds(j * IPG + t * L, L)] = g * mv[t]
                return carry2

            lax.fori_loop(0, G, rep_body, 0)

            pltpu.sync_copy(
                trows_v, out_hbm.at[s, :, pl.ds(b0 + half * C, C)]
            )
        return carry

    lax.fori_loop(0, CHUNKS, chunk_body, 0)


def kernel(ids, embedding_mat):
    ids_sb = ids.T.reshape(SEQ, BATCH // IPG, IPG)
    table_pairs = embedding_mat.T.reshape(2 * D, VOCAB // 2).T
    out_phys = _embed_lookup(ids_sb, table_pairs)
    return jnp.transpose(out_phys, (2, 0, 1))


# speedup vs baseline: 1.6432x; 1.6432x over previous
import functools

import jax
import jax.numpy as jnp
from jax import lax
from jax.experimental import pallas as pl
from jax.experimental.pallas import tpu as pltpu
from jax.experimental.pallas import tpu_sc as plsc

VOCAB = 1000000
D = 64
BATCH = 4096
SEQ = 200
B = BATCH * SEQ
PAD_ID = 0

NC = 2
NS = 16
L = 16
NW = NC * NS
BPW = B // NW
IPG = 128
C = 512
G = C // IPG
CHUNKS = BPW // C
PAIRS = CHUNKS // 2
IDROWS = BPW // IPG

_mesh = plsc.VectorSubcoreMesh(core_axis_name="c", subcore_axis_name="s")


@functools.partial(
    pl.kernel,
    out_type=jax.ShapeDtypeStruct((BATCH, SEQ, D), jnp.float32),
    mesh=_mesh,
    scratch_types=[
        pltpu.VMEM((IDROWS, IPG), jnp.int32),
        pltpu.VMEM((C, D), jnp.float32),
        pltpu.VMEM((C, D), jnp.float32),
        pltpu.SemaphoreType.DMA,
        pltpu.SemaphoreType.DMA,
        pltpu.SemaphoreType.DMA,
        pltpu.SemaphoreType.DMA,
    ],
    compiler_params=pltpu.CompilerParams(use_tc_tiling_on_sc=False),
)
def _embed_lookup(ids_hbm, table_hbm, out_hbm, idx_v, rows0, rows1,
                  sg0, sg1, sw0, sw1):
    wid = lax.axis_index("s") * NC + lax.axis_index("c")
    base = wid * BPW

    pltpu.sync_copy(ids_hbm.at[pl.ds(wid * IDROWS, IDROWS)], idx_v)

    def fire_gathers(k, rows_v, sg):
        for j in range(G):
            pltpu.async_copy(
                table_hbm.at[idx_v.at[k * G + j]],
                rows_v.at[pl.ds(j * IPG, IPG)],
                sg,
            )

    def drain_gathers(k, rows_v, sg):
        for j in range(G):
            pltpu.make_async_copy(
                table_hbm.at[idx_v.at[k * G + j]],
                rows_v.at[pl.ds(j * IPG, IPG)],
                sg,
            ).wait()

    def out_slice(k):
        flat0 = base + k * C
        s = flat0 >> 12
        b0 = flat0 & (BATCH - 1)
        return out_hbm.at[pl.ds(b0, C), s]

    def fire_write(k, rows_v, sw):
        pltpu.async_copy(rows_v, out_slice(k), sw)

    def drain_write(k, rows_v, sw):
        pltpu.make_async_copy(rows_v, out_slice(k), sw).wait()

    def fixup(k, rows_v):
        vs = [
            idx_v[k * G + j, pl.ds(t * L, L)]
            for j in range(G)
            for t in range(IPG // L)
        ]
        mn_vec = functools.reduce(jnp.minimum, vs)
        mn = functools.reduce(jnp.minimum, [mn_vec[i] for i in range(L)])

        @pl.when(mn == PAD_ID)
        def _fixup():
            def grp_body(g, c):
                jq = g // (IPG // L)
                tq = g % (IPG // L)
                idv = idx_v[k * G + jq, pl.ds(tq * L, L)]
                mvec = jnp.where(idv == PAD_ID, 0.0, 1.0).astype(jnp.float32)
                for rl in range(L):
                    f = mvec[rl]
                    row = g * L + rl
                    for cb in range(D // L):
                        sl = pl.ds(cb * L, L)
                        rows_v[row, sl] = rows_v[row, sl] * f
                return c

            lax.fori_loop(0, C // L, grp_body, 0)

    fire_gathers(0, rows0, sg0)
    fire_gathers(1, rows1, sg1)

    def pair_body(i, carry):
        a = 2 * i
        for (ko, rows_v, sg, sw) in ((0, rows0, sg0, sw0),
                                     (1, rows1, sg1, sw1)):
            k = a + ko
            drain_gathers(k, rows_v, sg)
            fixup(k, rows_v)
            fire_write(k, rows_v, sw)

            @pl.when(k + 2 < CHUNKS)
            def _refill(k=k, rows_v=rows_v, sg=sg, sw=sw):
                drain_write(k, rows_v, sw)
                fire_gathers(k + 2, rows_v, sg)

        return carry

    lax.fori_loop(0, PAIRS, pair_body, 0)

    drain_write(CHUNKS - 2, rows0, sw0)
    drain_write(CHUNKS - 1, rows1, sw1)


def kernel(ids, embedding_mat):
    ids_sb = ids.T.reshape(B // IPG, IPG)
    return _embed_lookup(ids_sb, embedding_mat)
